# Initial kernel scaffold; baseline (speedup 1.0000x reference)
import jax
import jax.numpy as jnp
from jax.experimental import pallas as pl


def kernel(h, coord, edge_index, edge_attr, Win, bin_, Wout, bout, We1, be1, We2, be2, Wn1, bn1, Wn2, bn2, Wc1, bc1, Wc2):
    raise NotImplementedError("write your pallas kernel here")


# speedup vs baseline: 2.8005x; 2.8005x over previous
import functools

import jax
import jax.numpy as jnp
from jax import lax
from jax.experimental import pallas as pl
from jax.experimental.pallas import tpu as pltpu
from jax.experimental.pallas import tpu_sc as plsc

N = 10000
E = 320000
D = 128
DE = 16
L = 4
CP = 16

NC = 2
NS = 16
NW = NC * NS
EW = E // NW
C = 80
NCHUNK = EW // C
RSUB = N // NS

BE = 4000
BN = 2000

f32 = jnp.float32


def _silu(x):
    return x * (1.0 / (1.0 + jnp.exp(-x)))


def _gather_body(row_hbm, col_hbm, a_hbm, b_hbm, cp_hbm,
                 g1_hbm, g2_hbm, g1c_hbm, g2c_hbm,
                 idx1, idx2, b1, b2, b1c, b2c, sem):
    wid = lax.axis_index("s") * NC + lax.axis_index("c")
    base = wid * EW

    @pl.loop(0, NCHUNK)
    def _(ci):
        off = base + ci * C
        pltpu.sync_copy(row_hbm.at[pl.ds(off, C)], idx1)
        pltpu.sync_copy(col_hbm.at[pl.ds(off, C)], idx2)
        c1 = pltpu.async_copy(a_hbm.at[idx1], b1, sem)
        c2 = pltpu.async_copy(b_hbm.at[idx2], b2, sem)
        c3 = pltpu.async_copy(cp_hbm.at[idx1], b1c, sem)
        c4 = pltpu.async_copy(cp_hbm.at[idx2], b2c, sem)
        c1.wait()
        c2.wait()
        c3.wait()
        c4.wait()
        pltpu.sync_copy(b1, g1_hbm.at[pl.ds(off, C)])
        pltpu.sync_copy(b2, g2_hbm.at[pl.ds(off, C)])
        pltpu.sync_copy(b1c, g1c_hbm.at[pl.ds(off, C)])
        pltpu.sync_copy(b2c, g2c_hbm.at[pl.ds(off, C)])


def _gather(row, col, a, b, cp):
    mesh = plsc.VectorSubcoreMesh(core_axis_name="c", subcore_axis_name="s")
    out_type = [
        jax.ShapeDtypeStruct((E, D), f32),
        jax.ShapeDtypeStruct((E, D), f32),
        jax.ShapeDtypeStruct((E, CP), f32),
        jax.ShapeDtypeStruct((E, CP), f32),
    ]
    scratch = [
        pltpu.VMEM((C,), jnp.int32),
        pltpu.VMEM((C,), jnp.int32),
        pltpu.VMEM((C, D), f32),
        pltpu.VMEM((C, D), f32),
        pltpu.VMEM((C, CP), f32),
        pltpu.VMEM((C, CP), f32),
        pltpu.SemaphoreType.DMA,
    ]
    return pl.kernel(_gather_body, out_type=out_type, mesh=mesh,
                     scratch_types=scratch,
                     compiler_params=pltpu.CompilerParams(
                         use_tc_tiling_on_sc=False))(row, col, a, b, cp)


def _scatter_body(row_hbm, m_hbm, mc_hbm, z_hbm, zc_hbm, p_hbm, pc_hbm,
                  idx, bm, bmc, acc, accc):
    c = lax.axis_index("c")
    s = lax.axis_index("s")
    r0 = s * RSUB
    pltpu.sync_copy(z_hbm.at[pl.ds(r0, RSUB)], acc.at[pl.ds(r0, RSUB)])
    pltpu.sync_copy(zc_hbm.at[pl.ds(r0, RSUB)], accc.at[pl.ds(r0, RSUB)])
    plsc.subcore_barrier()

    wid = s * NC + c
    base = wid * EW

    @pl.loop(0, NCHUNK)
    def _(ci):
        off = base + ci * C
        pltpu.sync_copy(row_hbm.at[pl.ds(off, C)], idx)
        pltpu.sync_copy(m_hbm.at[pl.ds(off, C)], bm)
        pltpu.sync_copy(mc_hbm.at[pl.ds(off, C)], bmc)
        pltpu.sync_copy(bm, acc.at[idx], add=True)
        pltpu.sync_copy(bmc, accc.at[idx], add=True)

    plsc.subcore_barrier()
    pltpu.sync_copy(acc.at[pl.ds(r0, RSUB)], p_hbm.at[c].at[pl.ds(r0, RSUB)])
    pltpu.sync_copy(accc.at[pl.ds(r0, RSUB)], pc_hbm.at[c].at[pl.ds(r0, RSUB)])


def _scatter(row, m, mc, z, zc):
    mesh = plsc.VectorSubcoreMesh(core_axis_name="c", subcore_axis_name="s")
    out_type = [
        jax.ShapeDtypeStruct((NC, N, D), f32),
        jax.ShapeDtypeStruct((NC, N, CP), f32),
    ]
    scratch = [
        pltpu.VMEM((C,), jnp.int32),
        pltpu.VMEM((C, D), f32),
        pltpu.VMEM((C, CP), f32),
        pltpu.VMEM_SHARED((N, D), f32),
        pltpu.VMEM_SHARED((N, CP), f32),
    ]
    return pl.kernel(_scatter_body, out_type=out_type, mesh=mesh,
                     scratch_types=scratch,
                     compiler_params=pltpu.CompilerParams(
                         use_tc_tiling_on_sc=False))(row, m, mc, z, zc)


def _mm_body(x, w, b, o):
    o[...] = jnp.dot(x[...], w[...], preferred_element_type=f32) + b[...]


def _matmul_bias(x, w, b):
    nb = N // BN
    return pl.pallas_call(
        _mm_body,
        grid=(nb,),
        in_specs=[
            pl.BlockSpec((BN, D), lambda i: (i, 0)),
            pl.BlockSpec((D, D), lambda i: (0, 0)),
            pl.BlockSpec((1, D), lambda i: (0, 0)),
        ],
        out_specs=pl.BlockSpec((BN, D), lambda i: (i, 0)),
        out_shape=jax.ShapeDtypeStruct((N, D), f32),
    )(x, w, b.reshape(1, D))


def _prep_body(h, ws, wd, b1, ao, bo):
    hv = h[...]
    ao[...] = jnp.dot(hv, ws[...], preferred_element_type=f32) + b1[...]
    bo[...] = jnp.dot(hv, wd[...], preferred_element_type=f32)


def _prep(h, ws, wd, b1):
    nb = N // BN
    return pl.pallas_call(
        _prep_body,
        grid=(nb,),
        in_specs=[
            pl.BlockSpec((BN, D), lambda i: (i, 0)),
            pl.BlockSpec((D, D), lambda i: (0, 0)),
            pl.BlockSpec((D, D), lambda i: (0, 0)),
            pl.BlockSpec((1, D), lambda i: (0, 0)),
        ],
        out_specs=[pl.BlockSpec((BN, D), lambda i: (i, 0)),
                   pl.BlockSpec((BN, D), lambda i: (i, 0))],
        out_shape=[jax.ShapeDtypeStruct((N, D), f32),
                   jax.ShapeDtypeStruct((N, D), f32)],
    )(h, ws, wd, b1)


def _edge_body(g1, g2, g1c, g2c, ea, wr, wat, we2, be2, wc1, bc1, wc2t,
               mo, mco):
    pre = g1[...] + g2[...]
    cdp = g1c[...] - g2c[...]
    radial = jnp.sum(cdp * cdp, axis=1, keepdims=True)
    pre = pre + radial * wr[...] + jnp.dot(ea[...], wat[...],
                                           preferred_element_type=f32)
    m = _silu(pre)
    m = _silu(jnp.dot(m, we2[...], preferred_element_type=f32) + be2[...])
    t = _silu(jnp.dot(m, wc1[...], preferred_element_type=f32) + bc1[...])
    phi = jnp.sum(t * wc2t[...], axis=1, keepdims=True)
    mo[...] = m
    trans = jnp.clip(cdp * phi, -100.0, 100.0)
    lane = lax.broadcasted_iota(jnp.int32, trans.shape, 1)
    mco[...] = jnp.where(lane == 3, 1.0, trans)


def _edge(g1, g2, g1c, g2c, ea, wr, wat, we2, be2, wc1, bc1, wc2t):
    nb = E // BE
    return pl.pallas_call(
        _edge_body,
        grid=(nb,),
        in_specs=[
            pl.BlockSpec((BE, D), lambda i: (i, 0)),
            pl.BlockSpec((BE, D), lambda i: (i, 0)),
            pl.BlockSpec((BE, CP), lambda i: (i, 0)),
            pl.BlockSpec((BE, CP), lambda i: (i, 0)),
            pl.BlockSpec((BE, DE), lambda i: (i, 0)),
            pl.BlockSpec((1, D), lambda i: (0, 0)),
            pl.BlockSpec((DE, D), lambda i: (0, 0)),
            pl.BlockSpec((D, D), lambda i: (0, 0)),
            pl.BlockSpec((1, D), lambda i: (0, 0)),
            pl.BlockSpec((D, D), lambda i: (0, 0)),
            pl.BlockSpec((1, D), lambda i: (0, 0)),
            pl.BlockSpec((1, D), lambda i: (0, 0)),
        ],
        out_specs=[pl.BlockSpec((BE, D), lambda i: (i, 0)),
                   pl.BlockSpec((BE, CP), lambda i: (i, 0))],
        out_shape=[jax.ShapeDtypeStruct((E, D), f32),
                   jax.ShapeDtypeStruct((E, CP), f32)],
    )(g1, g2, g1c, g2c, ea, wr, wat, we2, be2, wc1, bc1, wc2t)


def _node_body(p, pc, h, wh, wa, b1, w2, b2, ho, co):
    pv = p[...]
    pcv = pc[...]
    red = pv[0] + pv[1]
    redc = pcv[0] + pcv[1]
    cnt = jnp.maximum(redc[:, 3:4], 1.0)
    lane = lax.broadcasted_iota(jnp.int32, redc.shape, 1)
    co[...] = jnp.where(lane < 3, redc / cnt, 0.0)
    hv = h[...]
    o = _silu(jnp.dot(hv, wh[...], preferred_element_type=f32)
              + jnp.dot(red, wa[...], preferred_element_type=f32) + b1[...])
    ho[...] = hv + jnp.dot(o, w2[...], preferred_element_type=f32) + b2[...]


def _node(p, pc, h, wh, wa, b1, w2, b2):
    nb = N // BN
    return pl.pallas_call(
        _node_body,
        grid=(nb,),
        in_specs=[
            pl.BlockSpec((NC, BN, D), lambda i: (0, i, 0)),
            pl.BlockSpec((NC, BN, CP), lambda i: (0, i, 0)),
            pl.BlockSpec((BN, D), lambda i: (i, 0)),
            pl.BlockSpec((D, D), lambda i: (0, 0)),
            pl.BlockSpec((D, D), lambda i: (0, 0)),
            pl.BlockSpec((1, D), lambda i: (0, 0)),
            pl.BlockSpec((D, D), lambda i: (0, 0)),
            pl.BlockSpec((1, D), lambda i: (0, 0)),
        ],
        out_specs=[pl.BlockSpec((BN, D), lambda i: (i, 0)),
                   pl.BlockSpec((BN, CP), lambda i: (i, 0))],
        out_shape=[jax.ShapeDtypeStruct((N, D), f32),
                   jax.ShapeDtypeStruct((N, CP), f32)],
    )(p, pc, h, wh, wa, b1, w2, b2)


def kernel(h, coord, edge_index, edge_attr, Win, bin_, Wout, bout,
           We1, be1, We2, be2, Wn1, bn1, Wn2, bn2, Wc1, bc1, Wc2):
    row = edge_index[0]
    col = edge_index[1]
    coordpad = jnp.pad(coord, ((0, 0), (0, CP - 3)))
    z = jnp.zeros((N, D), f32)
    zc = jnp.zeros((N, CP), f32)

    h = _matmul_bias(h, Win, bin_)
    for l in range(L):
        ws = We1[l, :D]
        wd = We1[l, D:2 * D]
        wr = We1[l, 2 * D:2 * D + 1]
        wat = We1[l, 2 * D + 1:]
        a, b = _prep(h, ws, wd, be1[l].reshape(1, D))
        g1, g2, g1c, g2c = _gather(row, col, a, b, coordpad)
        m, mc = _edge(g1, g2, g1c, g2c, edge_attr, wr, wat, We2[l],
                      be2[l].reshape(1, D), Wc1[l], bc1[l].reshape(1, D),
                      Wc2[l].reshape(1, D))
        p, pc = _scatter(row, m, mc, z, zc)
        h, coordpad = _node(p, pc, h, Wn1[l, :D], Wn1[l, D:],
                            bn1[l].reshape(1, D), Wn2[l], bn2[l].reshape(1, D))
    h = _matmul_bias(h, Wout, bout)
    return h, coordpad[:, :3]


# speedup vs baseline: 3.6287x; 1.2957x over previous
import functools

import jax
import jax.numpy as jnp
from jax import lax
from jax.experimental import pallas as pl
from jax.experimental.pallas import tpu as pltpu
from jax.experimental.pallas import tpu_sc as plsc

N = 10000
E = 320000
D = 128
DE = 16
L = 4
CP = 16

NC = 2
NS = 16
NW = NC * NS
EW = E // NW
C = 80
NCHUNK = EW // C
RSUB = N // NS

BE = 4000
BN = 2000

f32 = jnp.float32


def _silu(x):
    return x * (1.0 / (1.0 + jnp.exp(-x)))


def _gather_body(row3_hbm, col3_hbm, a_hbm, b_hbm, cp_hbm,
                 g1_hbm, g2_hbm, g1c_hbm, g2c_hbm,
                 idxr, idxc, b1, b2, b1c, b2c,
                 isem, gsem0, gsem1, wsem0, wsem1):
    wid = lax.axis_index("s") * NC + lax.axis_index("c")
    base = wid * EW
    gsems = (gsem0, gsem1)
    wsems = (wsem0, wsem1)

    pltpu.async_copy(row3_hbm.at[wid], idxr, isem)
    pltpu.async_copy(col3_hbm.at[wid], idxc, isem)
    pltpu.make_async_copy(row3_hbm.at[wid], idxr, isem).wait()
    pltpu.make_async_copy(col3_hbm.at[wid], idxc, isem).wait()

    def issue_gather(cj, k):
        pltpu.async_copy(a_hbm.at[idxr.at[cj]], b1.at[k], gsems[k])
        pltpu.async_copy(b_hbm.at[idxc.at[cj]], b2.at[k], gsems[k])
        pltpu.async_copy(cp_hbm.at[idxr.at[cj]], b1c.at[k], gsems[k])
        pltpu.async_copy(cp_hbm.at[idxc.at[cj]], b2c.at[k], gsems[k])

    def wait_gather(k):
        pltpu.make_async_copy(a_hbm.at[idxr.at[0]], b1.at[k], gsems[k]).wait()
        pltpu.make_async_copy(b_hbm.at[idxc.at[0]], b2.at[k], gsems[k]).wait()
        pltpu.make_async_copy(cp_hbm.at[idxr.at[0]], b1c.at[k], gsems[k]).wait()
        pltpu.make_async_copy(cp_hbm.at[idxc.at[0]], b2c.at[k], gsems[k]).wait()

    def issue_wb(cj, k):
        off = base + cj * C
        pltpu.async_copy(b1.at[k], g1_hbm.at[pl.ds(off, C)], wsems[k])
        pltpu.async_copy(b2.at[k], g2_hbm.at[pl.ds(off, C)], wsems[k])
        pltpu.async_copy(b1c.at[k], g1c_hbm.at[pl.ds(off, C)], wsems[k])
        pltpu.async_copy(b2c.at[k], g2c_hbm.at[pl.ds(off, C)], wsems[k])

    def wait_wb(k):
        off = base
        pltpu.make_async_copy(b1.at[k], g1_hbm.at[pl.ds(off, C)],
                              wsems[k]).wait()
        pltpu.make_async_copy(b2.at[k], g2_hbm.at[pl.ds(off, C)],
                              wsems[k]).wait()
        pltpu.make_async_copy(b1c.at[k], g1c_hbm.at[pl.ds(off, C)],
                              wsems[k]).wait()
        pltpu.make_async_copy(b2c.at[k], g2c_hbm.at[pl.ds(off, C)],
                              wsems[k]).wait()

    issue_gather(0, 0)

    @pl.loop(0, NCHUNK - 1, step=2)
    def _(ci):
        for k in (0, 1):
            cj = ci + k
            wait_gather(k)
            issue_wb(cj, k)

            @pl.when(cj >= 1)
            def _():
                wait_wb(1 - k)

            issue_gather(cj + 1, 1 - k)

    wait_gather(0)
    issue_wb(NCHUNK - 1, 0)
    wait_wb(1)
    wait_wb(0)


def _gather(row3, col3, a, b, cp):
    mesh = plsc.VectorSubcoreMesh(core_axis_name="c", subcore_axis_name="s")
    out_type = [
        jax.ShapeDtypeStruct((E, D), f32),
        jax.ShapeDtypeStruct((E, D), f32),
        jax.ShapeDtypeStruct((E, CP), f32),
        jax.ShapeDtypeStruct((E, CP), f32),
    ]
    scratch = [
        pltpu.VMEM((NCHUNK, C), jnp.int32),
        pltpu.VMEM((NCHUNK, C), jnp.int32),
        pltpu.VMEM((2, C, D), f32),
        pltpu.VMEM((2, C, D), f32),
        pltpu.VMEM((2, C, CP), f32),
        pltpu.VMEM((2, C, CP), f32),
        pltpu.SemaphoreType.DMA,
        pltpu.SemaphoreType.DMA,
        pltpu.SemaphoreType.DMA,
        pltpu.SemaphoreType.DMA,
        pltpu.SemaphoreType.DMA,
    ]
    return pl.kernel(_gather_body, out_type=out_type, mesh=mesh,
                     scratch_types=scratch,
                     compiler_params=pltpu.CompilerParams(
                         use_tc_tiling_on_sc=False))(row3, col3, a, b, cp)


def _scatter_body(row3_hbm, m_hbm, mc_hbm, z_hbm, zc_hbm, p_hbm, pc_hbm,
                  idxv, bm, bmc, acc, accc, isem, lsem0, lsem1):
    c = lax.axis_index("c")
    s = lax.axis_index("s")
    r0 = s * RSUB
    wid = s * NC + c
    base = wid * EW
    lsems = (lsem0, lsem1)

    pltpu.async_copy(row3_hbm.at[wid], idxv, isem)
    pltpu.sync_copy(z_hbm.at[pl.ds(r0, RSUB)], acc.at[pl.ds(r0, RSUB)])
    pltpu.sync_copy(zc_hbm.at[pl.ds(r0, RSUB)], accc.at[pl.ds(r0, RSUB)])
    pltpu.make_async_copy(row3_hbm.at[wid], idxv, isem).wait()
    plsc.subcore_barrier()

    def issue_load(cj, k):
        off = base + cj * C
        pltpu.async_copy(m_hbm.at[pl.ds(off, C)], bm.at[k], lsems[k])
        pltpu.async_copy(mc_hbm.at[pl.ds(off, C)], bmc.at[k], lsems[k])

    def wait_load(k):
        pltpu.make_async_copy(m_hbm.at[pl.ds(base, C)], bm.at[k],
                              lsems[k]).wait()
        pltpu.make_async_copy(mc_hbm.at[pl.ds(base, C)], bmc.at[k],
                              lsems[k]).wait()

    issue_load(0, 0)

    @pl.loop(0, NCHUNK - 1, step=2)
    def _(ci):
        for k in (0, 1):
            cj = ci + k
            wait_load(k)
            issue_load(cj + 1, 1 - k)
            pltpu.sync_copy(bm.at[k], acc.at[idxv.at[cj]], add=True)
            pltpu.sync_copy(bmc.at[k], accc.at[idxv.at[cj]], add=True)

    wait_load(0)
    pltpu.sync_copy(bm.at[0], acc.at[idxv.at[NCHUNK - 1]], add=True)
    pltpu.sync_copy(bmc.at[0], accc.at[idxv.at[NCHUNK - 1]], add=True)

    plsc.subcore_barrier()
    pltpu.sync_copy(acc.at[pl.ds(r0, RSUB)], p_hbm.at[c].at[pl.ds(r0, RSUB)])
    pltpu.sync_copy(accc.at[pl.ds(r0, RSUB)], pc_hbm.at[c].at[pl.ds(r0, RSUB)])


def _scatter(row3, m, mc, z, zc):
    mesh = plsc.VectorSubcoreMesh(core_axis_name="c", subcore_axis_name="s")
    out_type = [
        jax.ShapeDtypeStruct((NC, N, D), f32),
        jax.ShapeDtypeStruct((NC, N, CP), f32),
    ]
    scratch = [
        pltpu.VMEM((NCHUNK, C), jnp.int32),
        pltpu.VMEM((2, C, D), f32),
        pltpu.VMEM((2, C, CP), f32),
        pltpu.VMEM_SHARED((N, D), f32),
        pltpu.VMEM_SHARED((N, CP), f32),
        pltpu.SemaphoreType.DMA,
        pltpu.SemaphoreType.DMA,
        pltpu.SemaphoreType.DMA,
    ]
    return pl.kernel(_scatter_body, out_type=out_type, mesh=mesh,
                     scratch_types=scratch,
                     compiler_params=pltpu.CompilerParams(
                         use_tc_tiling_on_sc=False))(row3, m, mc, z, zc)


def _mm_body(x, w, b, o):
    o[...] = jnp.dot(x[...], w[...], preferred_element_type=f32) + b[...]


def _matmul_bias(x, w, b):
    nb = N // BN
    return pl.pallas_call(
        _mm_body,
        grid=(nb,),
        in_specs=[
            pl.BlockSpec((BN, D), lambda i: (i, 0)),
            pl.BlockSpec((D, D), lambda i: (0, 0)),
            pl.BlockSpec((1, D), lambda i: (0, 0)),
        ],
        out_specs=pl.BlockSpec((BN, D), lambda i: (i, 0)),
        out_shape=jax.ShapeDtypeStruct((N, D), f32),
    )(x, w, b.reshape(1, D))


def _prep_body(h, ws, wd, b1, ao, bo):
    hv = h[...]
    ao[...] = jnp.dot(hv, ws[...], preferred_element_type=f32) + b1[...]
    bo[...] = jnp.dot(hv, wd[...], preferred_element_type=f32)


def _prep(h, ws, wd, b1):
    nb = N // BN
    return pl.pallas_call(
        _prep_body,
        grid=(nb,),
        in_specs=[
            pl.BlockSpec((BN, D), lambda i: (i, 0)),
            pl.BlockSpec((D, D), lambda i: (0, 0)),
            pl.BlockSpec((D, D), lambda i: (0, 0)),
            pl.BlockSpec((1, D), lambda i: (0, 0)),
        ],
        out_specs=[pl.BlockSpec((BN, D), lambda i: (i, 0)),
                   pl.BlockSpec((BN, D), lambda i: (i, 0))],
        out_shape=[jax.ShapeDtypeStruct((N, D), f32),
                   jax.ShapeDtypeStruct((N, D), f32)],
    )(h, ws, wd, b1)


def _edge_body(g1, g2, g1c, g2c, ea, wr, wat, we2, be2, wc1, bc1, wc2t,
               mo, mco):
    pre = g1[...] + g2[...]
    cdp = g1c[...] - g2c[...]
    radial = jnp.sum(cdp * cdp, axis=1, keepdims=True)
    pre = pre + radial * wr[...] + jnp.dot(ea[...], wat[...],
                                           preferred_element_type=f32)
    m = _silu(pre)
    m = _silu(jnp.dot(m, we2[...], preferred_element_type=f32) + be2[...])
    t = _silu(jnp.dot(m, wc1[...], preferred_element_type=f32) + bc1[...])
    phi = jnp.sum(t * wc2t[...], axis=1, keepdims=True)
    mo[...] = m
    trans = jnp.clip(cdp * phi, -100.0, 100.0)
    lane = lax.broadcasted_iota(jnp.int32, trans.shape, 1)
    mco[...] = jnp.where(lane == 3, 1.0, trans)


def _edge(g1, g2, g1c, g2c, ea, wr, wat, we2, be2, wc1, bc1, wc2t):
    nb = E // BE
    return pl.pallas_call(
        _edge_body,
        grid=(nb,),
        in_specs=[
            pl.BlockSpec((BE, D), lambda i: (i, 0)),
            pl.BlockSpec((BE, D), lambda i: (i, 0)),
            pl.BlockSpec((BE, CP), lambda i: (i, 0)),
            pl.BlockSpec((BE, CP), lambda i: (i, 0)),
            pl.BlockSpec((BE, DE), lambda i: (i, 0)),
            pl.BlockSpec((1, D), lambda i: (0, 0)),
            pl.BlockSpec((DE, D), lambda i: (0, 0)),
            pl.BlockSpec((D, D), lambda i: (0, 0)),
            pl.BlockSpec((1, D), lambda i: (0, 0)),
            pl.BlockSpec((D, D), lambda i: (0, 0)),
            pl.BlockSpec((1, D), lambda i: (0, 0)),
            pl.BlockSpec((1, D), lambda i: (0, 0)),
        ],
        out_specs=[pl.BlockSpec((BE, D), lambda i: (i, 0)),
                   pl.BlockSpec((BE, CP), lambda i: (i, 0))],
        out_shape=[jax.ShapeDtypeStruct((E, D), f32),
                   jax.ShapeDtypeStruct((E, CP), f32)],
    )(g1, g2, g1c, g2c, ea, wr, wat, we2, be2, wc1, bc1, wc2t)


def _node_body(p, pc, h, wh, wa, b1, w2, b2, ho, co):
    pv = p[...]
    pcv = pc[...]
    red = pv[0] + pv[1]
    redc = pcv[0] + pcv[1]
    cnt = jnp.maximum(redc[:, 3:4], 1.0)
    lane = lax.broadcasted_iota(jnp.int32, redc.shape, 1)
    co[...] = jnp.where(lane < 3, redc / cnt, 0.0)
    hv = h[...]
    o = _silu(jnp.dot(hv, wh[...], preferred_element_type=f32)
              + jnp.dot(red, wa[...], preferred_element_type=f32) + b1[...])
    ho[...] = hv + jnp.dot(o, w2[...], preferred_element_type=f32) + b2[...]


def _node(p, pc, h, wh, wa, b1, w2, b2):
    nb = N // BN
    return pl.pallas_call(
        _node_body,
        grid=(nb,),
        in_specs=[
            pl.BlockSpec((NC, BN, D), lambda i: (0, i, 0)),
            pl.BlockSpec((NC, BN, CP), lambda i: (0, i, 0)),
            pl.BlockSpec((BN, D), lambda i: (i, 0)),
            pl.BlockSpec((D, D), lambda i: (0, 0)),
            pl.BlockSpec((D, D), lambda i: (0, 0)),
            pl.BlockSpec((1, D), lambda i: (0, 0)),
            pl.BlockSpec((D, D), lambda i: (0, 0)),
            pl.BlockSpec((1, D), lambda i: (0, 0)),
        ],
        out_specs=[pl.BlockSpec((BN, D), lambda i: (i, 0)),
                   pl.BlockSpec((BN, CP), lambda i: (i, 0))],
        out_shape=[jax.ShapeDtypeStruct((N, D), f32),
                   jax.ShapeDtypeStruct((N, CP), f32)],
    )(p, pc, h, wh, wa, b1, w2, b2)


def kernel(h, coord, edge_index, edge_attr, Win, bin_, Wout, bout,
           We1, be1, We2, be2, Wn1, bn1, Wn2, bn2, Wc1, bc1, Wc2):
    row = edge_index[0]
    col = edge_index[1]
    row3 = row.reshape(NW, NCHUNK, C)
    col3 = col.reshape(NW, NCHUNK, C)
    coordpad = jnp.pad(coord, ((0, 0), (0, CP - 3)))
    z = jnp.zeros((N, D), f32)
    zc = jnp.zeros((N, CP), f32)

    h = _matmul_bias(h, Win, bin_)
    for l in range(L):
        ws = We1[l, :D]
        wd = We1[l, D:2 * D]
        wr = We1[l, 2 * D:2 * D + 1]
        wat = We1[l, 2 * D + 1:]
        a, b = _prep(h, ws, wd, be1[l].reshape(1, D))
        g1, g2, g1c, g2c = _gather(row3, col3, a, b, coordpad)
        m, mc = _edge(g1, g2, g1c, g2c, edge_attr, wr, wat, We2[l],
                      be2[l].reshape(1, D), Wc1[l], bc1[l].reshape(1, D),
                      Wc2[l].reshape(1, D))
        p, pc = _scatter(row3, m, mc, z, zc)
        h, coordpad = _node(p, pc, h, Wn1[l, :D], Wn1[l, D:],
                            bn1[l].reshape(1, D), Wn2[l], bn2[l].reshape(1, D))
    h = _matmul_bias(h, Wout, bout)
    return h, coordpad[:, :3]


# speedup vs baseline: 4.8670x; 1.3413x over previous
import jax
import jax.numpy as jnp
from jax import lax
from jax.experimental import pallas as pl
from jax.experimental.pallas import tpu as pltpu
from jax.experimental.pallas import tpu_sc as plsc

N = 10000
E = 320000
D = 128
DE = 16
L = 4
CP = 16

NC = 2
NS = 16
NW = NC * NS
EW = E // NW

H = 2
E2 = E // H
EW2 = E2 // NW

CG = 128
NFULL = EW2 // CG
TAIL = EW2 - NFULL * CG
TOFF = NFULL * CG

CS = 40
NCHUNK = EW2 // CS

RSUB = N // NS

BE = 4000
BN = 2000

f32 = jnp.float32


def _silu(x):
    return x * (1.0 / (1.0 + jnp.exp(-x)))


_SC_PARAMS = pltpu.CompilerParams(use_tc_tiling_on_sc=False)


def _sc_mesh():
    return plsc.VectorSubcoreMesh(core_axis_name="c", subcore_axis_name="s")


def _gather_body(row2_hbm, col2_hbm, a_hbm, b_hbm, cp_hbm,
                 g1_hbm, g2_hbm, g1c_hbm, g2c_hbm,
                 idxr, idxc, b1, b2, b1c, b2c,
                 isem, gsem0, gsem1, wsem0, wsem1):
    wid = lax.axis_index("s") * NC + lax.axis_index("c")
    base = wid * EW2
    gsems = (gsem0, gsem1)
    wsems = (wsem0, wsem1)

    pltpu.async_copy(row2_hbm.at[wid], idxr, isem)
    pltpu.async_copy(col2_hbm.at[wid], idxc, isem)
    pltpu.make_async_copy(row2_hbm.at[wid], idxr, isem).wait()
    pltpu.make_async_copy(col2_hbm.at[wid], idxc, isem).wait()

    tb1 = b1.at[0].at[pl.ds(0, TAIL)]
    tb2 = b2.at[0].at[pl.ds(0, TAIL)]
    tb1c = b1c.at[0].at[pl.ds(0, TAIL)]
    tb2c = b2c.at[0].at[pl.ds(0, TAIL)]
    tir = idxr.at[pl.ds(TOFF, TAIL)]
    tic = idxc.at[pl.ds(TOFF, TAIL)]
    pltpu.async_copy(a_hbm.at[tir], tb1, gsem0)
    pltpu.async_copy(b_hbm.at[tic], tb2, gsem0)
    pltpu.async_copy(cp_hbm.at[tir], tb1c, gsem0)
    pltpu.async_copy(cp_hbm.at[tic], tb2c, gsem0)
    pltpu.make_async_copy(a_hbm.at[tir], tb1, gsem0).wait()
    pltpu.make_async_copy(b_hbm.at[tic], tb2, gsem0).wait()
    pltpu.make_async_copy(cp_hbm.at[tir], tb1c, gsem0).wait()
    pltpu.make_async_copy(cp_hbm.at[tic], tb2c, gsem0).wait()
    toff = base + TOFF
    pltpu.sync_copy(tb1, g1_hbm.at[pl.ds(toff, TAIL)])
    pltpu.sync_copy(tb2, g2_hbm.at[pl.ds(toff, TAIL)])
    pltpu.sync_copy(tb1c, g1c_hbm.at[pl.ds(toff, TAIL), pl.ds(0, CP)])
    pltpu.sync_copy(tb2c, g2c_hbm.at[pl.ds(toff, TAIL), pl.ds(0, CP)])

    def issue_gather(cj, k):
        ir = idxr.at[pl.ds(cj * CG, CG)]
        ic = idxc.at[pl.ds(cj * CG, CG)]
        pltpu.async_copy(a_hbm.at[ir], b1.at[k], gsems[k])
        pltpu.async_copy(b_hbm.at[ic], b2.at[k], gsems[k])
        pltpu.async_copy(cp_hbm.at[ir], b1c.at[k], gsems[k])
        pltpu.async_copy(cp_hbm.at[ic], b2c.at[k], gsems[k])

    def wait_gather(k):
        ir = idxr.at[pl.ds(0, CG)]
        pltpu.make_async_copy(a_hbm.at[ir], b1.at[k], gsems[k]).wait()
        pltpu.make_async_copy(b_hbm.at[ir], b2.at[k], gsems[k]).wait()
        pltpu.make_async_copy(cp_hbm.at[ir], b1c.at[k], gsems[k]).wait()
        pltpu.make_async_copy(cp_hbm.at[ir], b2c.at[k], gsems[k]).wait()

    def issue_wb(cj, k):
        off = base + cj * CG
        pltpu.async_copy(b1.at[k], g1_hbm.at[pl.ds(off, CG)], wsems[k])
        pltpu.async_copy(b2.at[k], g2_hbm.at[pl.ds(off, CG)], wsems[k])
        pltpu.async_copy(b1c.at[k], g1c_hbm.at[pl.ds(off, CG), pl.ds(0, CP)],
                         wsems[k])
        pltpu.async_copy(b2c.at[k], g2c_hbm.at[pl.ds(off, CG), pl.ds(0, CP)],
                         wsems[k])

    def wait_wb(k):
        off = base
        pltpu.make_async_copy(b1.at[k], g1_hbm.at[pl.ds(off, CG)],
                              wsems[k]).wait()
        pltpu.make_async_copy(b2.at[k], g2_hbm.at[pl.ds(off, CG)],
                              wsems[k]).wait()
        pltpu.make_async_copy(b1c.at[k],
                              g1c_hbm.at[pl.ds(off, CG), pl.ds(0, CP)],
                              wsems[k]).wait()
        pltpu.make_async_copy(b2c.at[k],
                              g2c_hbm.at[pl.ds(off, CG), pl.ds(0, CP)],
                              wsems[k]).wait()

    issue_gather(0, 0)

    @pl.loop(0, NFULL - 1, step=2)
    def _(ci):
        for k in (0, 1):
            cj = ci + k
            wait_gather(k)
            issue_wb(cj, k)

            @pl.when(cj >= 1)
            def _():
                wait_wb(1 - k)

            @pl.when(cj < NFULL - 1)
            def _():
                issue_gather(cj + 1, 1 - k)

    wait_gather(0)
    issue_wb(NFULL - 1, 0)
    wait_wb(1)
    wait_wb(0)


def _gather(row2, col2, a, b, cp):
    out_type = [
        jax.ShapeDtypeStruct((E2, D), f32),
        jax.ShapeDtypeStruct((E2, D), f32),
        jax.ShapeDtypeStruct((E2, D), f32),
        jax.ShapeDtypeStruct((E2, D), f32),
    ]
    scratch = [
        pltpu.VMEM((EW2,), jnp.int32),
        pltpu.VMEM((EW2,), jnp.int32),
        pltpu.VMEM((2, CG, D), f32),
        pltpu.VMEM((2, CG, D), f32),
        pltpu.VMEM((2, CG, CP), f32),
        pltpu.VMEM((2, CG, CP), f32),
        pltpu.SemaphoreType.DMA,
        pltpu.SemaphoreType.DMA,
        pltpu.SemaphoreType.DMA,
        pltpu.SemaphoreType.DMA,
        pltpu.SemaphoreType.DMA,
    ]
    return pl.kernel(_gather_body, out_type=out_type, mesh=_sc_mesh(),
                     scratch_types=scratch,
                     compiler_params=_SC_PARAMS)(row2, col2, a, b, cp)


def _scatter_body(row3_hbm, m_hbm, mc_hbm, z_hbm, zc_hbm, p_hbm, pc_hbm,
                  idxv, bm, bmc, acc, accc, isem, lsem0, lsem1):
    c = lax.axis_index("c")
    s = lax.axis_index("s")
    r0 = s * RSUB
    wid = s * NC + c
    base = wid * EW2
    lsems = (lsem0, lsem1)

    pltpu.async_copy(row3_hbm.at[wid], idxv, isem)
    pltpu.sync_copy(z_hbm.at[pl.ds(r0, RSUB)], acc.at[pl.ds(r0, RSUB)])
    pltpu.sync_copy(zc_hbm.at[pl.ds(r0, RSUB)], accc.at[pl.ds(r0, RSUB)])
    pltpu.make_async_copy(row3_hbm.at[wid], idxv, isem).wait()
    plsc.subcore_barrier()

    def issue_load(cj, k):
        off = base + cj * CS
        pltpu.async_copy(m_hbm.at[pl.ds(off, CS)], bm.at[k], lsems[k])
        pltpu.async_copy(mc_hbm.at[pl.ds(off, CS), pl.ds(0, CP)], bmc.at[k],
                         lsems[k])

    def wait_load(k):
        pltpu.make_async_copy(m_hbm.at[pl.ds(base, CS)], bm.at[k],
                              lsems[k]).wait()
        pltpu.make_async_copy(mc_hbm.at[pl.ds(base, CS), pl.ds(0, CP)],
                              bmc.at[k], lsems[k]).wait()

    issue_load(0, 0)

    @pl.loop(0, NCHUNK - 1, step=2)
    def _(ci):
        for k in (0, 1):
            cj = ci + k
            wait_load(k)
            issue_load(cj + 1, 1 - k)
            pltpu.sync_copy(bm.at[k], acc.at[idxv.at[cj]], add=True)
            pltpu.sync_copy(bmc.at[k], accc.at[idxv.at[cj]], add=True)

    wait_load(0)
    pltpu.sync_copy(bm.at[0], acc.at[idxv.at[NCHUNK - 1]], add=True)
    pltpu.sync_copy(bmc.at[0], accc.at[idxv.at[NCHUNK - 1]], add=True)

    plsc.subcore_barrier()
    pltpu.sync_copy(acc.at[pl.ds(r0, RSUB)], p_hbm.at[c].at[pl.ds(r0, RSUB)])
    pltpu.sync_copy(accc.at[pl.ds(r0, RSUB)], pc_hbm.at[c].at[pl.ds(r0, RSUB)])


def _scatter(row3, m, mc, z, zc):
    out_type = [
        jax.ShapeDtypeStruct((NC, N, D), f32),
        jax.ShapeDtypeStruct((NC, N, CP), f32),
    ]
    scratch = [
        pltpu.VMEM((NCHUNK, CS), jnp.int32),
        pltpu.VMEM((2, CS, D), f32),
        pltpu.VMEM((2, CS, CP), f32),
        pltpu.VMEM_SHARED((N, D), f32),
        pltpu.VMEM_SHARED((N, CP), f32),
        pltpu.SemaphoreType.DMA,
        pltpu.SemaphoreType.DMA,
        pltpu.SemaphoreType.DMA,
    ]
    return pl.kernel(_scatter_body, out_type=out_type, mesh=_sc_mesh(),
                     scratch_types=scratch,
                     compiler_params=_SC_PARAMS)(row3, m, mc, z, zc)


def _mm_body(x, w, b, o):
    o[...] = jnp.dot(x[...], w[...], preferred_element_type=f32) + b[...]


def _matmul_bias(x, w, b):
    nb = N // BN
    return pl.pallas_call(
        _mm_body,
        grid=(nb,),
        in_specs=[
            pl.BlockSpec((BN, D), lambda i: (i, 0)),
            pl.BlockSpec((D, D), lambda i: (0, 0)),
            pl.BlockSpec((1, D), lambda i: (0, 0)),
        ],
        out_specs=pl.BlockSpec((BN, D), lambda i: (i, 0)),
        out_shape=jax.ShapeDtypeStruct((N, D), f32),
    )(x, w, b.reshape(1, D))


def _prep_body(h, ws, wd, b1, ao, bo):
    hv = h[...]
    ao[...] = jnp.dot(hv, ws[...], preferred_element_type=f32) + b1[...]
    bo[...] = jnp.dot(hv, wd[...], preferred_element_type=f32)


def _prep(h, ws, wd, b1):
    nb = N // BN
    return pl.pallas_call(
        _prep_body,
        grid=(nb,),
        in_specs=[
            pl.BlockSpec((BN, D), lambda i: (i, 0)),
            pl.BlockSpec((D, D), lambda i: (0, 0)),
            pl.BlockSpec((D, D), lambda i: (0, 0)),
            pl.BlockSpec((1, D), lambda i: (0, 0)),
        ],
        out_specs=[pl.BlockSpec((BN, D), lambda i: (i, 0)),
                   pl.BlockSpec((BN, D), lambda i: (i, 0))],
        out_shape=[jax.ShapeDtypeStruct((N, D), f32),
                   jax.ShapeDtypeStruct((N, D), f32)],
    )(h, ws, wd, b1)


def _edge_body(g1, g2, g1c, g2c, ea, wr, wat, we2, be2, wc1, bc1, wc2t,
               mo, mco):
    pre = g1[...] + g2[...]
    cdp = g1c[...][:, :CP] - g2c[...][:, :CP]
    radial = jnp.sum(cdp * cdp, axis=1, keepdims=True)
    pre = pre + radial * wr[...] + jnp.dot(ea[...], wat[...],
                                           preferred_element_type=f32)
    m = _silu(pre)
    m = _silu(jnp.dot(m, we2[...], preferred_element_type=f32) + be2[...])
    t = _silu(jnp.dot(m, wc1[...], preferred_element_type=f32) + bc1[...])
    phi = jnp.sum(t * wc2t[...], axis=1, keepdims=True)
    mo[...] = m
    trans = jnp.clip(cdp * phi, -100.0, 100.0)
    lane = lax.broadcasted_iota(jnp.int32, trans.shape, 1)
    trans = jnp.where(lane == 3, 1.0, trans)
    mco[...] = jnp.pad(trans, ((0, 0), (0, D - CP)))


def _edge(g1, g2, g1c, g2c, ea, wr, wat, we2, be2, wc1, bc1, wc2t):
    nb = E2 // BE
    return pl.pallas_call(
        _edge_body,
        grid=(nb,),
        in_specs=[
            pl.BlockSpec((BE, D), lambda i: (i, 0)),
            pl.BlockSpec((BE, D), lambda i: (i, 0)),
            pl.BlockSpec((BE, D), lambda i: (i, 0)),
            pl.BlockSpec((BE, D), lambda i: (i, 0)),
            pl.BlockSpec((BE, DE), lambda i: (i, 0)),
            pl.BlockSpec((1, D), lambda i: (0, 0)),
            pl.BlockSpec((DE, D), lambda i: (0, 0)),
            pl.BlockSpec((D, D), lambda i: (0, 0)),
            pl.BlockSpec((1, D), lambda i: (0, 0)),
            pl.BlockSpec((D, D), lambda i: (0, 0)),
            pl.BlockSpec((1, D), lambda i: (0, 0)),
            pl.BlockSpec((1, D), lambda i: (0, 0)),
        ],
        out_specs=[pl.BlockSpec((BE, D), lambda i: (i, 0)),
                   pl.BlockSpec((BE, D), lambda i: (i, 0))],
        out_shape=[jax.ShapeDtypeStruct((E2, D), f32),
                   jax.ShapeDtypeStruct((E2, D), f32)],
    )(g1, g2, g1c, g2c, ea, wr, wat, we2, be2, wc1, bc1, wc2t)


def _node_body(p, pc, q, qc, h, wh, wa, b1, w2, b2, ho, co):
    pv = p[...]
    pcv = pc[...]
    qv = q[...]
    qcv = qc[...]
    red = (pv[0] + pv[1]) + (qv[0] + qv[1])
    redc = (pcv[0] + pcv[1]) + (qcv[0] + qcv[1])
    cnt = jnp.maximum(redc[:, 3:4], 1.0)
    lane = lax.broadcasted_iota(jnp.int32, redc.shape, 1)
    co[...] = jnp.where(lane < 3, redc / cnt, 0.0)
    hv = h[...]
    o = _silu(jnp.dot(hv, wh[...], preferred_element_type=f32)
              + jnp.dot(red, wa[...], preferred_element_type=f32) + b1[...])
    ho[...] = hv + jnp.dot(o, w2[...], preferred_element_type=f32) + b2[...]


def _node(p, pc, q, qc, h, wh, wa, b1, w2, b2):
    nb = N // BN
    return pl.pallas_call(
        _node_body,
        grid=(nb,),
        in_specs=[
            pl.BlockSpec((NC, BN, D), lambda i: (0, i, 0)),
            pl.BlockSpec((NC, BN, CP), lambda i: (0, i, 0)),
            pl.BlockSpec((NC, BN, D), lambda i: (0, i, 0)),
            pl.BlockSpec((NC, BN, CP), lambda i: (0, i, 0)),
            pl.BlockSpec((BN, D), lambda i: (i, 0)),
            pl.BlockSpec((D, D), lambda i: (0, 0)),
            pl.BlockSpec((D, D), lambda i: (0, 0)),
            pl.BlockSpec((1, D), lambda i: (0, 0)),
            pl.BlockSpec((D, D), lambda i: (0, 0)),
            pl.BlockSpec((1, D), lambda i: (0, 0)),
        ],
        out_specs=[pl.BlockSpec((BN, D), lambda i: (i, 0)),
                   pl.BlockSpec((BN, CP), lambda i: (i, 0))],
        out_shape=[jax.ShapeDtypeStruct((N, D), f32),
                   jax.ShapeDtypeStruct((N, CP), f32)],
    )(p, pc, q, qc, h, wh, wa, b1, w2, b2)


def kernel(h, coord, edge_index, edge_attr, Win, bin_, Wout, bout,
           We1, be1, We2, be2, Wn1, bn1, Wn2, bn2, Wc1, bc1, Wc2):
    row = edge_index[0]
    col = edge_index[1]
    row2 = [row[i * E2:(i + 1) * E2].reshape(NW, EW2) for i in range(H)]
    col2 = [col[i * E2:(i + 1) * E2].reshape(NW, EW2) for i in range(H)]
    row3 = [row[i * E2:(i + 1) * E2].reshape(NW, NCHUNK, CS) for i in range(H)]
    eah = [edge_attr[i * E2:(i + 1) * E2] for i in range(H)]
    coordpad = jnp.pad(coord, ((0, 0), (0, CP - 3)))
    z = jnp.zeros((N, D), f32)
    zc = jnp.zeros((N, CP), f32)

    h = _matmul_bias(h, Win, bin_)
    for l in range(L):
        ws = We1[l, :D]
        wd = We1[l, D:2 * D]
        wr = We1[l, 2 * D:2 * D + 1]
        wat = We1[l, 2 * D + 1:]
        a, b = _prep(h, ws, wd, be1[l].reshape(1, D))
        gs = [_gather(row2[i], col2[i], a, b, coordpad) for i in range(H)]
        ms = [_edge(gs[i][0], gs[i][1], gs[i][2], gs[i][3], eah[i], wr, wat,
                    We2[l], be2[l].reshape(1, D), Wc1[l],
                    bc1[l].reshape(1, D), Wc2[l].reshape(1, D))
              for i in range(H)]
        ps = [_scatter(row3[i], ms[i][0], ms[i][1], z, zc) for i in range(H)]
        h, coordpad = _node(ps[0][0], ps[0][1], ps[1][0], ps[1][1], h,
                            Wn1[l, :D], Wn1[l, D:], bn1[l].reshape(1, D),
                            Wn2[l], bn2[l].reshape(1, D))
    h = _matmul_bias(h, Wout, bout)
    return h, coordpad[:, :3]


# speedup vs baseline: 4.9969x; 1.0267x over previous
import jax
import jax.numpy as jnp
from jax import lax
from jax.experimental import pallas as pl
from jax.experimental.pallas import tpu as pltpu
from jax.experimental.pallas import tpu_sc as plsc

N = 10000
E = 320000
D = 128
DE = 16
L = 4
CP = 16

NC = 2
NS = 16
NW = NC * NS
EW = E // NW

CG = 128
NFULL = EW // CG
TAIL = EW - NFULL * CG
TOFF = NFULL * CG

CS = 80
NCHUNK = EW // CS

RSUB = N // NS

BE = 4000
BN = 2000

f32 = jnp.float32


def _silu(x):
    return jax.nn.silu(x)


_SC_PARAMS = pltpu.CompilerParams(use_tc_tiling_on_sc=False)


def _sc_mesh():
    return plsc.VectorSubcoreMesh(core_axis_name="c", subcore_axis_name="s")


def _gather_body(row2_hbm, col2_hbm, a_hbm, b_hbm, cp_hbm,
                 g1_hbm, g2_hbm, g1c_hbm, g2c_hbm,
                 idxr, idxc, b1, b2, b1c, b2c,
                 isem, gsem0, gsem1, wsem0, wsem1):
    wid = lax.axis_index("s") * NC + lax.axis_index("c")
    base = wid * EW
    gsems = (gsem0, gsem1)
    wsems = (wsem0, wsem1)

    pltpu.async_copy(row2_hbm.at[wid], idxr, isem)
    pltpu.async_copy(col2_hbm.at[wid], idxc, isem)
    pltpu.make_async_copy(row2_hbm.at[wid], idxr, isem).wait()
    pltpu.make_async_copy(col2_hbm.at[wid], idxc, isem).wait()

    tb1 = b1.at[0].at[pl.ds(0, TAIL)]
    tb2 = b2.at[0].at[pl.ds(0, TAIL)]
    tb1c = b1c.at[0].at[pl.ds(0, TAIL)]
    tb2c = b2c.at[0].at[pl.ds(0, TAIL)]
    tir = idxr.at[pl.ds(TOFF, TAIL)]
    tic = idxc.at[pl.ds(TOFF, TAIL)]
    pltpu.async_copy(a_hbm.at[tir], tb1, gsem0)
    pltpu.async_copy(b_hbm.at[tic], tb2, gsem0)
    pltpu.async_copy(cp_hbm.at[tir], tb1c, gsem0)
    pltpu.async_copy(cp_hbm.at[tic], tb2c, gsem0)
    pltpu.make_async_copy(a_hbm.at[tir], tb1, gsem0).wait()
    pltpu.make_async_copy(b_hbm.at[tic], tb2, gsem0).wait()
    pltpu.make_async_copy(cp_hbm.at[tir], tb1c, gsem0).wait()
    pltpu.make_async_copy(cp_hbm.at[tic], tb2c, gsem0).wait()
    toff = base + TOFF
    pltpu.sync_copy(tb1, g1_hbm.at[pl.ds(toff, TAIL)])
    pltpu.sync_copy(tb2, g2_hbm.at[pl.ds(toff, TAIL)])
    pltpu.sync_copy(tb1c, g1c_hbm.at[pl.ds(toff, TAIL), pl.ds(0, CP)])
    pltpu.sync_copy(tb2c, g2c_hbm.at[pl.ds(toff, TAIL), pl.ds(0, CP)])

    def issue_gather(cj, k):
        ir = idxr.at[pl.ds(cj * CG, CG)]
        ic = idxc.at[pl.ds(cj * CG, CG)]
        pltpu.async_copy(a_hbm.at[ir], b1.at[k], gsems[k])
        pltpu.async_copy(b_hbm.at[ic], b2.at[k], gsems[k])
        pltpu.async_copy(cp_hbm.at[ir], b1c.at[k], gsems[k])
        pltpu.async_copy(cp_hbm.at[ic], b2c.at[k], gsems[k])

    def wait_gather(k):
        ir = idxr.at[pl.ds(0, CG)]
        pltpu.make_async_copy(a_hbm.at[ir], b1.at[k], gsems[k]).wait()
        pltpu.make_async_copy(b_hbm.at[ir], b2.at[k], gsems[k]).wait()
        pltpu.make_async_copy(cp_hbm.at[ir], b1c.at[k], gsems[k]).wait()
        pltpu.make_async_copy(cp_hbm.at[ir], b2c.at[k], gsems[k]).wait()

    def issue_wb(cj, k):
        off = base + cj * CG
        pltpu.async_copy(b1.at[k], g1_hbm.at[pl.ds(off, CG)], wsems[k])
        pltpu.async_copy(b2.at[k], g2_hbm.at[pl.ds(off, CG)], wsems[k])
        pltpu.async_copy(b1c.at[k], g1c_hbm.at[pl.ds(off, CG), pl.ds(0, CP)],
                         wsems[k])
        pltpu.async_copy(b2c.at[k], g2c_hbm.at[pl.ds(off, CG), pl.ds(0, CP)],
                         wsems[k])

    def wait_wb(k):
        off = base
        pltpu.make_async_copy(b1.at[k], g1_hbm.at[pl.ds(off, CG)],
                              wsems[k]).wait()
        pltpu.make_async_copy(b2.at[k], g2_hbm.at[pl.ds(off, CG)],
                              wsems[k]).wait()
        pltpu.make_async_copy(b1c.at[k],
                              g1c_hbm.at[pl.ds(off, CG), pl.ds(0, CP)],
                              wsems[k]).wait()
        pltpu.make_async_copy(b2c.at[k],
                              g2c_hbm.at[pl.ds(off, CG), pl.ds(0, CP)],
                              wsems[k]).wait()

    issue_gather(0, 0)

    @pl.loop(0, NFULL, step=2)
    def _(ci):
        for k in (0, 1):
            cj = ci + k
            wait_gather(k)
            issue_wb(cj, k)

            @pl.when(cj >= 1)
            def _():
                wait_wb(1 - k)

            @pl.when(cj < NFULL - 1)
            def _():
                issue_gather(cj + 1, 1 - k)

    wait_wb(1)


def _gather(row2, col2, a, b, cp):
    out_type = [
        jax.ShapeDtypeStruct((E, D), f32),
        jax.ShapeDtypeStruct((E, D), f32),
        jax.ShapeDtypeStruct((E, D), f32),
        jax.ShapeDtypeStruct((E, D), f32),
    ]
    scratch = [
        pltpu.VMEM((EW,), jnp.int32),
        pltpu.VMEM((EW,), jnp.int32),
        pltpu.VMEM((2, CG, D), f32),
        pltpu.VMEM((2, CG, D), f32),
        pltpu.VMEM((2, CG, CP), f32),
        pltpu.VMEM((2, CG, CP), f32),
        pltpu.SemaphoreType.DMA,
        pltpu.SemaphoreType.DMA,
        pltpu.SemaphoreType.DMA,
        pltpu.SemaphoreType.DMA,
        pltpu.SemaphoreType.DMA,
    ]
    return pl.kernel(_gather_body, out_type=out_type, mesh=_sc_mesh(),
                     scratch_types=scratch,
                     compiler_params=_SC_PARAMS)(row2, col2, a, b, cp)


def _scatter_body(row3_hbm, m_hbm, mc_hbm, z_hbm, zc_hbm, p_hbm, pc_hbm,
                  idxv, bm, bmc, acc, accc, isem, lsem0, lsem1):
    c = lax.axis_index("c")
    s = lax.axis_index("s")
    r0 = s * RSUB
    wid = s * NC + c
    base = wid * EW
    lsems = (lsem0, lsem1)

    pltpu.async_copy(row3_hbm.at[wid], idxv, isem)
    pltpu.sync_copy(z_hbm.at[pl.ds(r0, RSUB)], acc.at[pl.ds(r0, RSUB)])
    pltpu.sync_copy(zc_hbm.at[pl.ds(r0, RSUB)], accc.at[pl.ds(r0, RSUB)])
    pltpu.make_async_copy(row3_hbm.at[wid], idxv, isem).wait()
    plsc.subcore_barrier()

    def issue_load(cj, k):
        off = base + cj * CS
        pltpu.async_copy(m_hbm.at[pl.ds(off, CS)], bm.at[k], lsems[k])
        pltpu.async_copy(mc_hbm.at[pl.ds(off, CS), pl.ds(0, CP)], bmc.at[k],
                         lsems[k])

    def wait_load(k):
        pltpu.make_async_copy(m_hbm.at[pl.ds(base, CS)], bm.at[k],
                              lsems[k]).wait()
        pltpu.make_async_copy(mc_hbm.at[pl.ds(base, CS), pl.ds(0, CP)],
                              bmc.at[k], lsems[k]).wait()

    issue_load(0, 0)

    @pl.loop(0, NCHUNK - 1, step=2)
    def _(ci):
        for k in (0, 1):
            cj = ci + k
            wait_load(k)
            issue_load(cj + 1, 1 - k)
            pltpu.sync_copy(bm.at[k], acc.at[idxv.at[cj]], add=True)
            pltpu.sync_copy(bmc.at[k], accc.at[idxv.at[cj]], add=True)

    wait_load(0)
    pltpu.sync_copy(bm.at[0], acc.at[idxv.at[NCHUNK - 1]], add=True)
    pltpu.sync_copy(bmc.at[0], accc.at[idxv.at[NCHUNK - 1]], add=True)

    plsc.subcore_barrier()
    pltpu.sync_copy(acc.at[pl.ds(r0, RSUB)], p_hbm.at[c].at[pl.ds(r0, RSUB)])
    pltpu.sync_copy(accc.at[pl.ds(r0, RSUB)], pc_hbm.at[c].at[pl.ds(r0, RSUB)])


def _scatter(row3, m, mc, z, zc):
    out_type = [
        jax.ShapeDtypeStruct((NC, N, D), f32),
        jax.ShapeDtypeStruct((NC, N, CP), f32),
    ]
    scratch = [
        pltpu.VMEM((NCHUNK, CS), jnp.int32),
        pltpu.VMEM((2, CS, D), f32),
        pltpu.VMEM((2, CS, CP), f32),
        pltpu.VMEM_SHARED((N, D), f32),
        pltpu.VMEM_SHARED((N, CP), f32),
        pltpu.SemaphoreType.DMA,
        pltpu.SemaphoreType.DMA,
        pltpu.SemaphoreType.DMA,
    ]
    return pl.kernel(_scatter_body, out_type=out_type, mesh=_sc_mesh(),
                     scratch_types=scratch,
                     compiler_params=_SC_PARAMS)(row3, m, mc, z, zc)


def _mm_body(x, w, b, o):
    o[...] = jnp.dot(x[...], w[...], preferred_element_type=f32) + b[...]


def _matmul_bias(x, w, b):
    nb = N // BN
    return pl.pallas_call(
        _mm_body,
        grid=(nb,),
        in_specs=[
            pl.BlockSpec((BN, D), lambda i: (i, 0)),
            pl.BlockSpec((D, D), lambda i: (0, 0)),
            pl.BlockSpec((1, D), lambda i: (0, 0)),
        ],
        out_specs=pl.BlockSpec((BN, D), lambda i: (i, 0)),
        out_shape=jax.ShapeDtypeStruct((N, D), f32),
    )(x, w, b.reshape(1, D))


def _prep_body(h, ws, wd, b1, ao, bo):
    hv = h[...]
    ao[...] = jnp.dot(hv, ws[...], preferred_element_type=f32) + b1[...]
    bo[...] = jnp.dot(hv, wd[...], preferred_element_type=f32)


def _prep(h, ws, wd, b1):
    nb = N // BN
    return pl.pallas_call(
        _prep_body,
        grid=(nb,),
        in_specs=[
            pl.BlockSpec((BN, D), lambda i: (i, 0)),
            pl.BlockSpec((D, D), lambda i: (0, 0)),
            pl.BlockSpec((D, D), lambda i: (0, 0)),
            pl.BlockSpec((1, D), lambda i: (0, 0)),
        ],
        out_specs=[pl.BlockSpec((BN, D), lambda i: (i, 0)),
                   pl.BlockSpec((BN, D), lambda i: (i, 0))],
        out_shape=[jax.ShapeDtypeStruct((N, D), f32),
                   jax.ShapeDtypeStruct((N, D), f32)],
    )(h, ws, wd, b1)


def _edge_body(g1, g2, g1c, g2c, ea, wr, wat, we2, be2, wc1, bc1, wc2t,
               mo, mco):
    pre = g1[...] + g2[...]
    cdp = g1c[...][:, :CP] - g2c[...][:, :CP]
    radial = jnp.sum(cdp * cdp, axis=1, keepdims=True)
    pre = pre + radial * wr[...] + jnp.dot(ea[...], wat[...],
                                           preferred_element_type=f32)
    m = _silu(pre)
    m = _silu(jnp.dot(m, we2[...], preferred_element_type=f32) + be2[...])
    t = _silu(jnp.dot(m, wc1[...], preferred_element_type=f32) + bc1[...])
    phi = jnp.sum(t * wc2t[...], axis=1, keepdims=True)
    mo[...] = m
    trans = jnp.clip(cdp * phi, -100.0, 100.0)
    lane = lax.broadcasted_iota(jnp.int32, trans.shape, 1)
    trans = jnp.where(lane == 3, 1.0, trans)
    mco[...] = jnp.pad(trans, ((0, 0), (0, D - CP)))


def _edge(g1, g2, g1c, g2c, ea, wr, wat, we2, be2, wc1, bc1, wc2t):
    nb = E // BE
    return pl.pallas_call(
        _edge_body,
        grid=(nb,),
        in_specs=[
            pl.BlockSpec((BE, D), lambda i: (i, 0)),
            pl.BlockSpec((BE, D), lambda i: (i, 0)),
            pl.BlockSpec((BE, D), lambda i: (i, 0)),
            pl.BlockSpec((BE, D), lambda i: (i, 0)),
            pl.BlockSpec((BE, DE), lambda i: (i, 0)),
            pl.BlockSpec((1, D), lambda i: (0, 0)),
            pl.BlockSpec((DE, D), lambda i: (0, 0)),
            pl.BlockSpec((D, D), lambda i: (0, 0)),
            pl.BlockSpec((1, D), lambda i: (0, 0)),
            pl.BlockSpec((D, D), lambda i: (0, 0)),
            pl.BlockSpec((1, D), lambda i: (0, 0)),
            pl.BlockSpec((1, D), lambda i: (0, 0)),
        ],
        out_specs=[pl.BlockSpec((BE, D), lambda i: (i, 0)),
                   pl.BlockSpec((BE, D), lambda i: (i, 0))],
        out_shape=[jax.ShapeDtypeStruct((E, D), f32),
                   jax.ShapeDtypeStruct((E, D), f32)],
    )(g1, g2, g1c, g2c, ea, wr, wat, we2, be2, wc1, bc1, wc2t)


def _node_body(p, pc, h, wh, wa, b1, w2, b2, ho, co):
    pv = p[...]
    pcv = pc[...]
    red = pv[0] + pv[1]
    redc = pcv[0] + pcv[1]
    cnt = jnp.maximum(redc[:, 3:4], 1.0)
    lane = lax.broadcasted_iota(jnp.int32, redc.shape, 1)
    co[...] = jnp.where(lane < 3, redc / cnt, 0.0)
    hv = h[...]
    o = _silu(jnp.dot(hv, wh[...], preferred_element_type=f32)
              + jnp.dot(red, wa[...], preferred_element_type=f32) + b1[...])
    ho[...] = hv + jnp.dot(o, w2[...], preferred_element_type=f32) + b2[...]


def _node(p, pc, h, wh, wa, b1, w2, b2):
    nb = N // BN
    return pl.pallas_call(
        _node_body,
        grid=(nb,),
        in_specs=[
            pl.BlockSpec((NC, BN, D), lambda i: (0, i, 0)),
            pl.BlockSpec((NC, BN, CP), lambda i: (0, i, 0)),
            pl.BlockSpec((BN, D), lambda i: (i, 0)),
            pl.BlockSpec((D, D), lambda i: (0, 0)),
            pl.BlockSpec((D, D), lambda i: (0, 0)),
            pl.BlockSpec((1, D), lambda i: (0, 0)),
            pl.BlockSpec((D, D), lambda i: (0, 0)),
            pl.BlockSpec((1, D), lambda i: (0, 0)),
        ],
        out_specs=[pl.BlockSpec((BN, D), lambda i: (i, 0)),
                   pl.BlockSpec((BN, CP), lambda i: (i, 0))],
        out_shape=[jax.ShapeDtypeStruct((N, D), f32),
                   jax.ShapeDtypeStruct((N, CP), f32)],
    )(p, pc, h, wh, wa, b1, w2, b2)


def kernel(h, coord, edge_index, edge_attr, Win, bin_, Wout, bout,
           We1, be1, We2, be2, Wn1, bn1, Wn2, bn2, Wc1, bc1, Wc2):
    row = edge_index[0]
    col = edge_index[1]
    row2 = row.reshape(NW, EW)
    col2 = col.reshape(NW, EW)
    row3 = row.reshape(NW, NCHUNK, CS)
    coordpad = jnp.pad(coord, ((0, 0), (0, CP - 3)))
    z = jnp.zeros((N, D), f32)
    zc = jnp.zeros((N, CP), f32)

    h = _matmul_bias(h, Win, bin_)
    for l in range(L):
        ws = We1[l, :D]
        wd = We1[l, D:2 * D]
        wr = We1[l, 2 * D:2 * D + 1]
        wat = We1[l, 2 * D + 1:]
        a, b = _prep(h, ws, wd, be1[l].reshape(1, D))
        g1, g2, g1c, g2c = _gather(row2, col2, a, b, coordpad)
        m, mc = _edge(g1, g2, g1c, g2c, edge_attr, wr, wat, We2[l],
                      be2[l].reshape(1, D), Wc1[l], bc1[l].reshape(1, D),
                      Wc2[l].reshape(1, D))
        p, pc = _scatter(row3, m, mc, z, zc)
        h, coordpad = _node(p, pc, h, Wn1[l, :D], Wn1[l, D:],
                            bn1[l].reshape(1, D), Wn2[l], bn2[l].reshape(1, D))
    h = _matmul_bias(h, Wout, bout)
    return h, coordpad[:, :3]


# speedup vs baseline: 5.0057x; 1.0018x over previous
import jax
import jax.numpy as jnp
from jax import lax
from jax.experimental import pallas as pl
from jax.experimental.pallas import tpu as pltpu
from jax.experimental.pallas import tpu_sc as plsc

N = 10000
E = 320000
D = 128
DE = 16
L = 4
CP = 16

NC = 2
NS = 16
NW = NC * NS
EW = E // NW

CG = 128
NFULL = EW // CG
TAIL = EW - NFULL * CG
TOFF = NFULL * CG

CS = 80
NCHUNK = EW // CS

RSUB = N // NS

BE = 6400
BN = 2000

f32 = jnp.float32


def _silu(x):
    return jax.nn.silu(x)


_SC_PARAMS = pltpu.CompilerParams(use_tc_tiling_on_sc=False)


def _sc_mesh():
    return plsc.VectorSubcoreMesh(core_axis_name="c", subcore_axis_name="s")


def _gather_body(row2_hbm, col2_hbm, a_hbm, b_hbm, cp_hbm,
                 g1_hbm, g2_hbm, gd_hbm,
                 idxr, idxc, b1, b2, b1c, b2c,
                 isem, gsem0, gsem1, wsem0, wsem1):
    wid = lax.axis_index("s") * NC + lax.axis_index("c")
    base = wid * EW
    gsems = (gsem0, gsem1)
    wsems = (wsem0, wsem1)

    pltpu.async_copy(row2_hbm.at[wid], idxr, isem)
    pltpu.async_copy(col2_hbm.at[wid], idxc, isem)
    pltpu.make_async_copy(row2_hbm.at[wid], idxr, isem).wait()
    pltpu.make_async_copy(col2_hbm.at[wid], idxc, isem).wait()

    tb1 = b1.at[0].at[pl.ds(0, TAIL)]
    tb2 = b2.at[0].at[pl.ds(0, TAIL)]
    tb1c = b1c.at[0].at[pl.ds(0, TAIL)]
    tb2c = b2c.at[0].at[pl.ds(0, TAIL)]
    tir = idxr.at[pl.ds(TOFF, TAIL)]
    tic = idxc.at[pl.ds(TOFF, TAIL)]
    pltpu.async_copy(a_hbm.at[tir], tb1, gsem0)
    pltpu.async_copy(b_hbm.at[tic], tb2, gsem0)
    pltpu.async_copy(cp_hbm.at[tir], tb1c, gsem0)
    pltpu.async_copy(cp_hbm.at[tic], tb2c, gsem0)
    pltpu.make_async_copy(a_hbm.at[tir], tb1, gsem0).wait()
    pltpu.make_async_copy(b_hbm.at[tic], tb2, gsem0).wait()
    pltpu.make_async_copy(cp_hbm.at[tir], tb1c, gsem0).wait()
    pltpu.make_async_copy(cp_hbm.at[tic], tb2c, gsem0).wait()

    @pl.loop(0, TAIL)
    def _(r):
        b1c[0, r, :] = b1c[0, r, :] - b2c[0, r, :]

    toff = base + TOFF
    pltpu.sync_copy(tb1, g1_hbm.at[pl.ds(toff, TAIL)])
    pltpu.sync_copy(tb2, g2_hbm.at[pl.ds(toff, TAIL)])
    pltpu.sync_copy(tb1c, gd_hbm.at[pl.ds(toff, TAIL), pl.ds(0, CP)])

    def issue_gather(cj, k):
        ir = idxr.at[pl.ds(cj * CG, CG)]
        ic = idxc.at[pl.ds(cj * CG, CG)]
        pltpu.async_copy(a_hbm.at[ir], b1.at[k], gsems[k])
        pltpu.async_copy(b_hbm.at[ic], b2.at[k], gsems[k])
        pltpu.async_copy(cp_hbm.at[ir], b1c.at[k], gsems[k])
        pltpu.async_copy(cp_hbm.at[ic], b2c.at[k], gsems[k])

    def wait_gather(k):
        ir = idxr.at[pl.ds(0, CG)]
        pltpu.make_async_copy(a_hbm.at[ir], b1.at[k], gsems[k]).wait()
        pltpu.make_async_copy(b_hbm.at[ir], b2.at[k], gsems[k]).wait()
        pltpu.make_async_copy(cp_hbm.at[ir], b1c.at[k], gsems[k]).wait()
        pltpu.make_async_copy(cp_hbm.at[ir], b2c.at[k], gsems[k]).wait()

    def issue_wb(cj, k):
        off = base + cj * CG
        pltpu.async_copy(b1.at[k], g1_hbm.at[pl.ds(off, CG)], wsems[k])
        pltpu.async_copy(b2.at[k], g2_hbm.at[pl.ds(off, CG)], wsems[k])
        pltpu.async_copy(b1c.at[k], gd_hbm.at[pl.ds(off, CG), pl.ds(0, CP)],
                         wsems[k])

    def wait_wb(k):
        off = base
        pltpu.make_async_copy(b1.at[k], g1_hbm.at[pl.ds(off, CG)],
                              wsems[k]).wait()
        pltpu.make_async_copy(b2.at[k], g2_hbm.at[pl.ds(off, CG)],
                              wsems[k]).wait()
        pltpu.make_async_copy(b1c.at[k],
                              gd_hbm.at[pl.ds(off, CG), pl.ds(0, CP)],
                              wsems[k]).wait()

    issue_gather(0, 0)

    @pl.loop(0, NFULL, step=2)
    def _(ci):
        for k in (0, 1):
            cj = ci + k
            wait_gather(k)

            @pl.loop(0, CG)
            def _(r):
                b1c[k, r, :] = b1c[k, r, :] - b2c[k, r, :]

            issue_wb(cj, k)

            @pl.when(cj >= 1)
            def _():
                wait_wb(1 - k)

            @pl.when(cj < NFULL - 1)
            def _():
                issue_gather(cj + 1, 1 - k)

    wait_wb(1)


def _gather(row2, col2, a, b, cp):
    out_type = [
        jax.ShapeDtypeStruct((E, D), f32),
        jax.ShapeDtypeStruct((E, D), f32),
        jax.ShapeDtypeStruct((E, D), f32),
    ]
    scratch = [
        pltpu.VMEM((EW,), jnp.int32),
        pltpu.VMEM((EW,), jnp.int32),
        pltpu.VMEM((2, CG, D), f32),
        pltpu.VMEM((2, CG, D), f32),
        pltpu.VMEM((2, CG, CP), f32),
        pltpu.VMEM((2, CG, CP), f32),
        pltpu.SemaphoreType.DMA,
        pltpu.SemaphoreType.DMA,
        pltpu.SemaphoreType.DMA,
        pltpu.SemaphoreType.DMA,
        pltpu.SemaphoreType.DMA,
    ]
    return pl.kernel(_gather_body, out_type=out_type, mesh=_sc_mesh(),
                     scratch_types=scratch,
                     compiler_params=_SC_PARAMS)(row2, col2, a, b, cp)


def _scatter_body(row3_hbm, m_hbm, mc_hbm, z_hbm, zc_hbm, p_hbm, pc_hbm,
                  idxv, bm, bmc, acc, accc, isem, lsem0, lsem1):
    c = lax.axis_index("c")
    s = lax.axis_index("s")
    r0 = s * RSUB
    wid = s * NC + c
    base = wid * EW
    lsems = (lsem0, lsem1)

    pltpu.async_copy(row3_hbm.at[wid], idxv, isem)
    pltpu.sync_copy(z_hbm.at[pl.ds(r0, RSUB)], acc.at[pl.ds(r0, RSUB)])
    pltpu.sync_copy(zc_hbm.at[pl.ds(r0, RSUB)], accc.at[pl.ds(r0, RSUB)])
    pltpu.make_async_copy(row3_hbm.at[wid], idxv, isem).wait()
    plsc.subcore_barrier()

    def issue_load(cj, k):
        off = base + cj * CS
        pltpu.async_copy(m_hbm.at[pl.ds(off, CS)], bm.at[k], lsems[k])
        pltpu.async_copy(mc_hbm.at[pl.ds(off, CS), pl.ds(0, CP)], bmc.at[k],
                         lsems[k])

    def wait_load(k):
        pltpu.make_async_copy(m_hbm.at[pl.ds(base, CS)], bm.at[k],
                              lsems[k]).wait()
        pltpu.make_async_copy(mc_hbm.at[pl.ds(base, CS), pl.ds(0, CP)],
                              bmc.at[k], lsems[k]).wait()

    issue_load(0, 0)

    @pl.loop(0, NCHUNK - 1, step=2)
    def _(ci):
        for k in (0, 1):
            cj = ci + k
            wait_load(k)
            issue_load(cj + 1, 1 - k)
            pltpu.sync_copy(bm.at[k], acc.at[idxv.at[cj]], add=True)
            pltpu.sync_copy(bmc.at[k], accc.at[idxv.at[cj]], add=True)

    wait_load(0)
    pltpu.sync_copy(bm.at[0], acc.at[idxv.at[NCHUNK - 1]], add=True)
    pltpu.sync_copy(bmc.at[0], accc.at[idxv.at[NCHUNK - 1]], add=True)

    plsc.subcore_barrier()
    pltpu.sync_copy(acc.at[pl.ds(r0, RSUB)], p_hbm.at[c].at[pl.ds(r0, RSUB)])
    pltpu.sync_copy(accc.at[pl.ds(r0, RSUB)], pc_hbm.at[c].at[pl.ds(r0, RSUB)])


def _scatter(row3, m, mc, z, zc):
    out_type = [
        jax.ShapeDtypeStruct((NC, N, D), f32),
        jax.ShapeDtypeStruct((NC, N, CP), f32),
    ]
    scratch = [
        pltpu.VMEM((NCHUNK, CS), jnp.int32),
        pltpu.VMEM((2, CS, D), f32),
        pltpu.VMEM((2, CS, CP), f32),
        pltpu.VMEM_SHARED((N, D), f32),
        pltpu.VMEM_SHARED((N, CP), f32),
        pltpu.SemaphoreType.DMA,
        pltpu.SemaphoreType.DMA,
        pltpu.SemaphoreType.DMA,
    ]
    return pl.kernel(_scatter_body, out_type=out_type, mesh=_sc_mesh(),
                     scratch_types=scratch,
                     compiler_params=_SC_PARAMS)(row3, m, mc, z, zc)


def _mm_body(x, w, b, o):
    o[...] = jnp.dot(x[...], w[...], preferred_element_type=f32) + b[...]


def _matmul_bias(x, w, b):
    nb = N // BN
    return pl.pallas_call(
        _mm_body,
        grid=(nb,),
        in_specs=[
            pl.BlockSpec((BN, D), lambda i: (i, 0)),
            pl.BlockSpec((D, D), lambda i: (0, 0)),
            pl.BlockSpec((1, D), lambda i: (0, 0)),
        ],
        out_specs=pl.BlockSpec((BN, D), lambda i: (i, 0)),
        out_shape=jax.ShapeDtypeStruct((N, D), f32),
    )(x, w, b.reshape(1, D))


def _prep_body(h, ws, wd, b1, ao, bo):
    hv = h[...]
    ao[...] = jnp.dot(hv, ws[...], preferred_element_type=f32) + b1[...]
    bo[...] = jnp.dot(hv, wd[...], preferred_element_type=f32)


def _prep(h, ws, wd, b1):
    nb = N // BN
    return pl.pallas_call(
        _prep_body,
        grid=(nb,),
        in_specs=[
            pl.BlockSpec((BN, D), lambda i: (i, 0)),
            pl.BlockSpec((D, D), lambda i: (0, 0)),
            pl.BlockSpec((D, D), lambda i: (0, 0)),
            pl.BlockSpec((1, D), lambda i: (0, 0)),
        ],
        out_specs=[pl.BlockSpec((BN, D), lambda i: (i, 0)),
                   pl.BlockSpec((BN, D), lambda i: (i, 0))],
        out_shape=[jax.ShapeDtypeStruct((N, D), f32),
                   jax.ShapeDtypeStruct((N, D), f32)],
    )(h, ws, wd, b1)


def _edge_body(g1, g2, gd, ea, wr, wat, we2, be2, wc1, bc1, wc2t,
               mo, mco):
    pre = g1[...] + g2[...]
    cdp = gd[...][:, :CP]
    radial = jnp.sum(cdp * cdp, axis=1, keepdims=True)
    pre = pre + radial * wr[...] + jnp.dot(ea[...], wat[...],
                                           preferred_element_type=f32)
    m = _silu(pre)
    m = _silu(jnp.dot(m, we2[...], preferred_element_type=f32) + be2[...])
    t = _silu(jnp.dot(m, wc1[...], preferred_element_type=f32) + bc1[...])
    phi = jnp.sum(t * wc2t[...], axis=1, keepdims=True)
    mo[...] = m
    trans = jnp.clip(cdp * phi, -100.0, 100.0)
    lane = lax.broadcasted_iota(jnp.int32, trans.shape, 1)
    trans = jnp.where(lane == 3, 1.0, trans)
    mco[...] = jnp.pad(trans, ((0, 0), (0, D - CP)))


def _edge(g1, g2, gd, ea, wr, wat, we2, be2, wc1, bc1, wc2t):
    nb = E // BE
    return pl.pallas_call(
        _edge_body,
        grid=(nb,),
        in_specs=[
            pl.BlockSpec((BE, D), lambda i: (i, 0)),
            pl.BlockSpec((BE, D), lambda i: (i, 0)),
            pl.BlockSpec((BE, D), lambda i: (i, 0)),
            pl.BlockSpec((BE, DE), lambda i: (i, 0)),
            pl.BlockSpec((1, D), lambda i: (0, 0)),
            pl.BlockSpec((DE, D), lambda i: (0, 0)),
            pl.BlockSpec((D, D), lambda i: (0, 0)),
            pl.BlockSpec((1, D), lambda i: (0, 0)),
            pl.BlockSpec((D, D), lambda i: (0, 0)),
            pl.BlockSpec((1, D), lambda i: (0, 0)),
            pl.BlockSpec((1, D), lambda i: (0, 0)),
        ],
        out_specs=[pl.BlockSpec((BE, D), lambda i: (i, 0)),
                   pl.BlockSpec((BE, D), lambda i: (i, 0))],
        out_shape=[jax.ShapeDtypeStruct((E, D), f32),
                   jax.ShapeDtypeStruct((E, D), f32)],
    )(g1, g2, gd, ea, wr, wat, we2, be2, wc1, bc1, wc2t)


def _node_body(p, pc, h, wh, wa, b1, w2, b2, ho, co):
    pv = p[...]
    pcv = pc[...]
    red = pv[0] + pv[1]
    redc = pcv[0] + pcv[1]
    cnt = jnp.maximum(redc[:, 3:4], 1.0)
    lane = lax.broadcasted_iota(jnp.int32, redc.shape, 1)
    co[...] = jnp.where(lane < 3, redc / cnt, 0.0)
    hv = h[...]
    o = _silu(jnp.dot(hv, wh[...], preferred_element_type=f32)
              + jnp.dot(red, wa[...], preferred_element_type=f32) + b1[...])
    ho[...] = hv + jnp.dot(o, w2[...], preferred_element_type=f32) + b2[...]


def _node(p, pc, h, wh, wa, b1, w2, b2):
    nb = N // BN
    return pl.pallas_call(
        _node_body,
        grid=(nb,),
        in_specs=[
            pl.BlockSpec((NC, BN, D), lambda i: (0, i, 0)),
            pl.BlockSpec((NC, BN, CP), lambda i: (0, i, 0)),
            pl.BlockSpec((BN, D), lambda i: (i, 0)),
            pl.BlockSpec((D, D), lambda i: (0, 0)),
            pl.BlockSpec((D, D), lambda i: (0, 0)),
            pl.BlockSpec((1, D), lambda i: (0, 0)),
            pl.BlockSpec((D, D), lambda i: (0, 0)),
            pl.BlockSpec((1, D), lambda i: (0, 0)),
        ],
        out_specs=[pl.BlockSpec((BN, D), lambda i: (i, 0)),
                   pl.BlockSpec((BN, CP), lambda i: (i, 0))],
        out_shape=[jax.ShapeDtypeStruct((N, D), f32),
                   jax.ShapeDtypeStruct((N, CP), f32)],
    )(p, pc, h, wh, wa, b1, w2, b2)


def kernel(h, coord, edge_index, edge_attr, Win, bin_, Wout, bout,
           We1, be1, We2, be2, Wn1, bn1, Wn2, bn2, Wc1, bc1, Wc2):
    row = edge_index[0]
    col = edge_index[1]
    row2 = row.reshape(NW, EW)
    col2 = col.reshape(NW, EW)
    row3 = row.reshape(NW, NCHUNK, CS)
    coordpad = jnp.pad(coord, ((0, 0), (0, CP - 3)))
    z = jnp.zeros((N, D), f32)
    zc = jnp.zeros((N, CP), f32)

    h = _matmul_bias(h, Win, bin_)
    for l in range(L):
        ws = We1[l, :D]
        wd = We1[l, D:2 * D]
        wr = We1[l, 2 * D:2 * D + 1]
        wat = We1[l, 2 * D + 1:]
        a, b = _prep(h, ws, wd, be1[l].reshape(1, D))
        g1, g2, gd = _gather(row2, col2, a, b, coordpad)
        m, mc = _edge(g1, g2, gd, edge_attr, wr, wat, We2[l],
                      be2[l].reshape(1, D), Wc1[l], bc1[l].reshape(1, D),
                      Wc2[l].reshape(1, D))
        p, pc = _scatter(row3, m, mc, z, zc)
        h, coordpad = _node(p, pc, h, Wn1[l, :D], Wn1[l, D:],
                            bn1[l].reshape(1, D), Wn2[l], bn2[l].reshape(1, D))
    h = _matmul_bias(h, Wout, bout)
    return h, coordpad[:, :3]


# speedup vs baseline: 5.2624x; 1.0513x over previous
import jax
import jax.numpy as jnp
from jax import lax
from jax.experimental import pallas as pl
from jax.experimental.pallas import tpu as pltpu
from jax.experimental.pallas import tpu_sc as plsc

N = 10000
E = 320000
D = 128
DE = 16
L = 4
CP = 16

NC = 2
NS = 16
NW = NC * NS
EW = E // NW

CG = 128
NFULL = EW // CG
TAIL = EW - NFULL * CG
TOFF = NFULL * CG

CS = 80
NCHUNK = EW // CS

RSUB = N // NS

BE = 6400
BN = 2000

f32 = jnp.float32


def _silu(x):
    return jax.nn.silu(x)


_SC_PARAMS = pltpu.CompilerParams(use_tc_tiling_on_sc=False)


def _sc_mesh():
    return plsc.VectorSubcoreMesh(core_axis_name="c", subcore_axis_name="s")


def _gather_body(row2_hbm, col2_hbm, a_hbm, b_hbm, cp_hbm,
                 g1_hbm, g2_hbm, gd_hbm,
                 idxr, idxc, b1, b2, b1c, b2c,
                 isem, gsem0, gsem1, wsem0, wsem1):
    wid = lax.axis_index("s") * NC + lax.axis_index("c")
    base = wid * EW
    gsems = (gsem0, gsem1)
    wsems = (wsem0, wsem1)

    pltpu.async_copy(row2_hbm.at[wid], idxr, isem)
    pltpu.async_copy(col2_hbm.at[wid], idxc, isem)
    pltpu.make_async_copy(row2_hbm.at[wid], idxr, isem).wait()
    pltpu.make_async_copy(col2_hbm.at[wid], idxc, isem).wait()

    tb1 = b1.at[0].at[pl.ds(0, TAIL)]
    tb2 = b2.at[0].at[pl.ds(0, TAIL)]
    tb1c = b1c.at[0].at[pl.ds(0, TAIL)]
    tb2c = b2c.at[0].at[pl.ds(0, TAIL)]
    tir = idxr.at[pl.ds(TOFF, TAIL)]
    tic = idxc.at[pl.ds(TOFF, TAIL)]
    pltpu.async_copy(a_hbm.at[tir], tb1, gsem0)
    pltpu.async_copy(b_hbm.at[tic], tb2, gsem0)
    pltpu.async_copy(cp_hbm.at[tir], tb1c, gsem0)
    pltpu.async_copy(cp_hbm.at[tic], tb2c, gsem0)
    pltpu.make_async_copy(a_hbm.at[tir], tb1, gsem0).wait()
    pltpu.make_async_copy(b_hbm.at[tic], tb2, gsem0).wait()
    pltpu.make_async_copy(cp_hbm.at[tir], tb1c, gsem0).wait()
    pltpu.make_async_copy(cp_hbm.at[tic], tb2c, gsem0).wait()

    @pl.loop(0, TAIL)
    def _(r):
        b1c[0, r, :] = b1c[0, r, :] - b2c[0, r, :]

    toff = base + TOFF
    pltpu.sync_copy(tb1, g1_hbm.at[pl.ds(toff, TAIL)])
    pltpu.sync_copy(tb2, g2_hbm.at[pl.ds(toff, TAIL)])
    pltpu.sync_copy(tb1c, gd_hbm.at[pl.ds(toff, TAIL), pl.ds(0, CP)])

    def issue_gather(cj, k):
        ir = idxr.at[pl.ds(cj * CG, CG)]
        ic = idxc.at[pl.ds(cj * CG, CG)]
        pltpu.async_copy(a_hbm.at[ir], b1.at[k], gsems[k])
        pltpu.async_copy(b_hbm.at[ic], b2.at[k], gsems[k])
        pltpu.async_copy(cp_hbm.at[ir], b1c.at[k], gsems[k])
        pltpu.async_copy(cp_hbm.at[ic], b2c.at[k], gsems[k])

    def wait_gather(k):
        ir = idxr.at[pl.ds(0, CG)]
        pltpu.make_async_copy(a_hbm.at[ir], b1.at[k], gsems[k]).wait()
        pltpu.make_async_copy(b_hbm.at[ir], b2.at[k], gsems[k]).wait()
        pltpu.make_async_copy(cp_hbm.at[ir], b1c.at[k], gsems[k]).wait()
        pltpu.make_async_copy(cp_hbm.at[ir], b2c.at[k], gsems[k]).wait()

    def issue_wb(cj, k):
        off = base + cj * CG
        pltpu.async_copy(b1.at[k], g1_hbm.at[pl.ds(off, CG)], wsems[k])
        pltpu.async_copy(b2.at[k], g2_hbm.at[pl.ds(off, CG)], wsems[k])
        pltpu.async_copy(b1c.at[k], gd_hbm.at[pl.ds(off, CG), pl.ds(0, CP)],
                         wsems[k])

    def wait_wb(k):
        off = base
        pltpu.make_async_copy(b1.at[k], g1_hbm.at[pl.ds(off, CG)],
                              wsems[k]).wait()
        pltpu.make_async_copy(b2.at[k], g2_hbm.at[pl.ds(off, CG)],
                              wsems[k]).wait()
        pltpu.make_async_copy(b1c.at[k],
                              gd_hbm.at[pl.ds(off, CG), pl.ds(0, CP)],
                              wsems[k]).wait()

    issue_gather(0, 0)

    @pl.loop(0, NFULL, step=2)
    def _(ci):
        for k in (0, 1):
            cj = ci + k
            wait_gather(k)

            @pl.loop(0, CG)
            def _(r):
                b1c[k, r, :] = b1c[k, r, :] - b2c[k, r, :]

            issue_wb(cj, k)

            @pl.when(cj >= 1)
            def _():
                wait_wb(1 - k)

            @pl.when(cj < NFULL - 1)
            def _():
                issue_gather(cj + 1, 1 - k)

    wait_wb(1)


def _gather(row2, col2, a, b, cp):
    out_type = [
        jax.ShapeDtypeStruct((E, D), f32),
        jax.ShapeDtypeStruct((E, D), f32),
        jax.ShapeDtypeStruct((E, D), f32),
    ]
    scratch = [
        pltpu.VMEM((EW,), jnp.int32),
        pltpu.VMEM((EW,), jnp.int32),
        pltpu.VMEM((2, CG, D), f32),
        pltpu.VMEM((2, CG, D), f32),
        pltpu.VMEM((2, CG, CP), f32),
        pltpu.VMEM((2, CG, CP), f32),
        pltpu.SemaphoreType.DMA,
        pltpu.SemaphoreType.DMA,
        pltpu.SemaphoreType.DMA,
        pltpu.SemaphoreType.DMA,
        pltpu.SemaphoreType.DMA,
    ]
    return pl.kernel(_gather_body, out_type=out_type, mesh=_sc_mesh(),
                     scratch_types=scratch,
                     compiler_params=_SC_PARAMS)(row2, col2, a, b, cp)


def _scatter_body(row3_hbm, m_hbm, mc_hbm, z_hbm, zc_hbm, p_hbm, pc_hbm,
                  idxv, bm, bmc, acc, accc, isem, lsem0, lsem1):
    c = lax.axis_index("c")
    s = lax.axis_index("s")
    r0 = s * RSUB
    wid = s * NC + c
    base = wid * EW
    lsems = (lsem0, lsem1)

    pltpu.async_copy(row3_hbm.at[wid], idxv, isem)
    pltpu.sync_copy(z_hbm.at[pl.ds(r0, RSUB)], acc.at[pl.ds(r0, RSUB)])
    pltpu.sync_copy(zc_hbm.at[pl.ds(r0, RSUB)], accc.at[pl.ds(r0, RSUB)])
    pltpu.make_async_copy(row3_hbm.at[wid], idxv, isem).wait()
    plsc.subcore_barrier()

    def issue_load(cj, k):
        off = base + cj * CS
        pltpu.async_copy(m_hbm.at[pl.ds(off, CS)], bm.at[k], lsems[k])
        pltpu.async_copy(mc_hbm.at[pl.ds(off, CS), pl.ds(0, CP)], bmc.at[k],
                         lsems[k])

    def wait_load(k):
        pltpu.make_async_copy(m_hbm.at[pl.ds(base, CS)], bm.at[k],
                              lsems[k]).wait()
        pltpu.make_async_copy(mc_hbm.at[pl.ds(base, CS), pl.ds(0, CP)],
                              bmc.at[k], lsems[k]).wait()

    issue_load(0, 0)

    @pl.loop(0, NCHUNK - 1, step=2)
    def _(ci):
        for k in (0, 1):
            cj = ci + k
            wait_load(k)
            issue_load(cj + 1, 1 - k)
            pltpu.sync_copy(bm.at[k], acc.at[idxv.at[cj]], add=True)
            pltpu.sync_copy(bmc.at[k], accc.at[idxv.at[cj]], add=True)

    wait_load(0)
    pltpu.sync_copy(bm.at[0], acc.at[idxv.at[NCHUNK - 1]], add=True)
    pltpu.sync_copy(bmc.at[0], accc.at[idxv.at[NCHUNK - 1]], add=True)

    plsc.subcore_barrier()
    pltpu.sync_copy(acc.at[pl.ds(r0, RSUB)], p_hbm.at[c].at[pl.ds(r0, RSUB)])
    pltpu.sync_copy(accc.at[pl.ds(r0, RSUB)], pc_hbm.at[c].at[pl.ds(r0, RSUB)])


def _scatter(row3, m, mc, z, zc):
    out_type = [
        jax.ShapeDtypeStruct((NC, N, D), f32),
        jax.ShapeDtypeStruct((NC, N, CP), f32),
    ]
    scratch = [
        pltpu.VMEM((NCHUNK, CS), jnp.int32),
        pltpu.VMEM((2, CS, D), f32),
        pltpu.VMEM((2, CS, CP), f32),
        pltpu.VMEM_SHARED((N, D), f32),
        pltpu.VMEM_SHARED((N, CP), f32),
        pltpu.SemaphoreType.DMA,
        pltpu.SemaphoreType.DMA,
        pltpu.SemaphoreType.DMA,
    ]
    return pl.kernel(_scatter_body, out_type=out_type, mesh=_sc_mesh(),
                     scratch_types=scratch,
                     compiler_params=_SC_PARAMS)(row3, m, mc, z, zc)


def _mm_body(x, w, b, o):
    o[...] = jnp.dot(x[...], w[...], preferred_element_type=f32) + b[...]


def _matmul_bias(x, w, b):
    nb = N // BN
    return pl.pallas_call(
        _mm_body,
        grid=(nb,),
        in_specs=[
            pl.BlockSpec((BN, D), lambda i: (i, 0)),
            pl.BlockSpec((D, D), lambda i: (0, 0)),
            pl.BlockSpec((1, D), lambda i: (0, 0)),
        ],
        out_specs=pl.BlockSpec((BN, D), lambda i: (i, 0)),
        out_shape=jax.ShapeDtypeStruct((N, D), f32),
    )(x, w, b.reshape(1, D))


def _prep_body(h, ws, wd, b1, ao, bo):
    hv = h[...]
    ao[...] = jnp.dot(hv, ws[...], preferred_element_type=f32) + b1[...]
    bo[...] = jnp.dot(hv, wd[...], preferred_element_type=f32)


def _prep(h, ws, wd, b1):
    nb = N // BN
    return pl.pallas_call(
        _prep_body,
        grid=(nb,),
        in_specs=[
            pl.BlockSpec((BN, D), lambda i: (i, 0)),
            pl.BlockSpec((D, D), lambda i: (0, 0)),
            pl.BlockSpec((D, D), lambda i: (0, 0)),
            pl.BlockSpec((1, D), lambda i: (0, 0)),
        ],
        out_specs=[pl.BlockSpec((BN, D), lambda i: (i, 0)),
                   pl.BlockSpec((BN, D), lambda i: (i, 0))],
        out_shape=[jax.ShapeDtypeStruct((N, D), f32),
                   jax.ShapeDtypeStruct((N, D), f32)],
    )(h, ws, wd, b1)


def _edge_body(g1, g2, gd, ea, wr, wat, we2, be2, wc1, bc1, wc2t,
               mo, mco):
    pre = g1[...] + g2[...]
    cdp = gd[...][:, :CP]
    radial = jnp.sum(cdp * cdp, axis=1, keepdims=True)
    eac = lax.dot_general(ea[...], wat[...], (((0,), (0,)), ((), ())),
                          preferred_element_type=f32)
    pre = pre + radial * wr[...] + eac
    m = _silu(pre)
    m = _silu(jnp.dot(m, we2[...], preferred_element_type=f32) + be2[...])
    t = _silu(jnp.dot(m, wc1[...], preferred_element_type=f32) + bc1[...])
    phi = jnp.sum(t * wc2t[...], axis=1, keepdims=True)
    mo[...] = m
    trans = jnp.clip(cdp * phi, -100.0, 100.0)
    lane = lax.broadcasted_iota(jnp.int32, trans.shape, 1)
    trans = jnp.where(lane == 3, 1.0, trans)
    mco[...] = jnp.pad(trans, ((0, 0), (0, D - CP)))


def _edge(g1, g2, gd, ea, wr, wat, we2, be2, wc1, bc1, wc2t):
    nb = E // BE
    return pl.pallas_call(
        _edge_body,
        grid=(nb,),
        in_specs=[
            pl.BlockSpec((BE, D), lambda i: (i, 0)),
            pl.BlockSpec((BE, D), lambda i: (i, 0)),
            pl.BlockSpec((BE, D), lambda i: (i, 0)),
            pl.BlockSpec((DE, BE), lambda i: (0, i)),
            pl.BlockSpec((1, D), lambda i: (0, 0)),
            pl.BlockSpec((DE, D), lambda i: (0, 0)),
            pl.BlockSpec((D, D), lambda i: (0, 0)),
            pl.BlockSpec((1, D), lambda i: (0, 0)),
            pl.BlockSpec((D, D), lambda i: (0, 0)),
            pl.BlockSpec((1, D), lambda i: (0, 0)),
            pl.BlockSpec((1, D), lambda i: (0, 0)),
        ],
        out_specs=[pl.BlockSpec((BE, D), lambda i: (i, 0)),
                   pl.BlockSpec((BE, D), lambda i: (i, 0))],
        out_shape=[jax.ShapeDtypeStruct((E, D), f32),
                   jax.ShapeDtypeStruct((E, D), f32)],
    )(g1, g2, gd, ea, wr, wat, we2, be2, wc1, bc1, wc2t)


def _node_body(p, pc, h, wh, wa, b1, w2, b2, ho, co):
    pv = p[...]
    pcv = pc[...]
    red = pv[0] + pv[1]
    redc = pcv[0] + pcv[1]
    cnt = jnp.maximum(redc[:, 3:4], 1.0)
    lane = lax.broadcasted_iota(jnp.int32, redc.shape, 1)
    co[...] = jnp.where(lane < 3, redc / cnt, 0.0)
    hv = h[...]
    o = _silu(jnp.dot(hv, wh[...], preferred_element_type=f32)
              + jnp.dot(red, wa[...], preferred_element_type=f32) + b1[...])
    ho[...] = hv + jnp.dot(o, w2[...], preferred_element_type=f32) + b2[...]


def _node(p, pc, h, wh, wa, b1, w2, b2):
    nb = N // BN
    return pl.pallas_call(
        _node_body,
        grid=(nb,),
        in_specs=[
            pl.BlockSpec((NC, BN, D), lambda i: (0, i, 0)),
            pl.BlockSpec((NC, BN, CP), lambda i: (0, i, 0)),
            pl.BlockSpec((BN, D), lambda i: (i, 0)),
            pl.BlockSpec((D, D), lambda i: (0, 0)),
            pl.BlockSpec((D, D), lambda i: (0, 0)),
            pl.BlockSpec((1, D), lambda i: (0, 0)),
            pl.BlockSpec((D, D), lambda i: (0, 0)),
            pl.BlockSpec((1, D), lambda i: (0, 0)),
        ],
        out_specs=[pl.BlockSpec((BN, D), lambda i: (i, 0)),
                   pl.BlockSpec((BN, CP), lambda i: (i, 0))],
        out_shape=[jax.ShapeDtypeStruct((N, D), f32),
                   jax.ShapeDtypeStruct((N, CP), f32)],
    )(p, pc, h, wh, wa, b1, w2, b2)


def kernel(h, coord, edge_index, edge_attr, Win, bin_, Wout, bout,
           We1, be1, We2, be2, Wn1, bn1, Wn2, bn2, Wc1, bc1, Wc2):
    row = edge_index[0]
    col = edge_index[1]
    row2 = row.reshape(NW, EW)
    col2 = col.reshape(NW, EW)
    row3 = row.reshape(NW, NCHUNK, CS)
    eat = edge_attr.T
    coordpad = jnp.pad(coord, ((0, 0), (0, CP - 3)))
    z = jnp.zeros((N, D), f32)
    zc = jnp.zeros((N, CP), f32)

    h = _matmul_bias(h, Win, bin_)
    for l in range(L):
        ws = We1[l, :D]
        wd = We1[l, D:2 * D]
        wr = We1[l, 2 * D:2 * D + 1]
        wat = We1[l, 2 * D + 1:]
        a, b = _prep(h, ws, wd, be1[l].reshape(1, D))
        g1, g2, gd = _gather(row2, col2, a, b, coordpad)
        m, mc = _edge(g1, g2, gd, eat, wr, wat, We2[l],
                      be2[l].reshape(1, D), Wc1[l], bc1[l].reshape(1, D),
                      Wc2[l].reshape(1, D))
        p, pc = _scatter(row3, m, mc, z, zc)
        h, coordpad = _node(p, pc, h, Wn1[l, :D], Wn1[l, D:],
                            bn1[l].reshape(1, D), Wn2[l], bn2[l].reshape(1, D))
    h = _matmul_bias(h, Wout, bout)
    return h, coordpad[:, :3]
